# baseline (speedup 1.0000x reference)
import functools

import jax
import jax.numpy as jnp
from jax import lax
from jax.experimental import pallas as pl
from jax.experimental.pallas import tpu as pltpu
from jax.experimental.pallas import tpu_sc as plsc

N = 10000
D = 128
EPS = 1e-5

NC = 2
NS = 16
NW = NC * NS
K = 128
RCH = 80
NRCH = N // RCH
HPAD = 16


GB = 1


def _seg_sum_partials(h, sd_pairs, zrows, ch, ch0, ch1):
    mesh = plsc.VectorSubcoreMesh(core_axis_name="c", subcore_axis_name="s",
                                  num_cores=NC, num_subcores=NS)

    @functools.partial(
        pl.kernel,
        out_type=jax.ShapeDtypeStruct((NC, N, D), jnp.float32),
        mesh=mesh,
        scratch_types=[
            pltpu.VMEM((K, D), jnp.float32),
            pltpu.VMEM((2, ch, K), jnp.int32),
            pltpu.VMEM_SHARED((N, D), jnp.float32),
            pltpu.SemaphoreType.DMA,
        ],
    )
    def k(h_hbm, sd_hbm, z_hbm, out_hbm, rows, sd, acc, gsem):
        cid = lax.axis_index("c")
        sid = lax.axis_index("s")
        wid = cid * NS + sid

        pltpu.sync_copy(sd_hbm.at[wid * 2], sd.at[0])
        pltpu.sync_copy(sd_hbm.at[wid * 2 + 1], sd.at[1])

        @pl.loop(sid, NRCH, step=NS)
        def _(q):
            pltpu.sync_copy(z_hbm, acc.at[pl.ds(q * RCH, RCH)])

        plsc.subcore_barrier()
        chw = jnp.where(cid == 0, ch0, ch1)
        pltpu.async_copy(h_hbm.at[sd.at[0, 0]], rows, gsem)

        @pl.loop(0, chw)
        def _(i):
            pltpu.make_async_copy(h_hbm.at[sd.at[0, i]], rows, gsem).wait()
            pltpu.sync_copy(rows, acc.at[sd.at[1, i]], add=True)

            @pl.when(i + 1 < chw)
            def _():
                pltpu.async_copy(h_hbm.at[sd.at[0, i + 1]], rows, gsem)

        plsc.subcore_barrier()

        @pl.loop(sid, NRCH, step=NS)
        def _(q):
            pltpu.sync_copy(acc.at[pl.ds(q * RCH, RCH)],
                            out_hbm.at[cid, pl.ds(q * RCH, RCH)])

    return k(h, sd_pairs, zrows)


def _tc_matmul(x, w):
    def body(x_ref, w_ref, o_ref):
        o_ref[...] = jnp.dot(x_ref[...], w_ref[...],
                             preferred_element_type=jnp.float32,
                             precision=lax.Precision.HIGHEST)

    return pl.pallas_call(
        body, out_shape=jax.ShapeDtypeStruct((N, D), jnp.float32))(x, w)


def _tc_bn_relu_matmul(parts, gamma, beta, mean, var, w):
    def body(p_ref, g_ref, b_ref, m_ref, v_ref, w_ref, o_ref):
        s = p_ref[0] + p_ref[1]
        scale = g_ref[...] * lax.rsqrt(v_ref[...] + EPS)
        shift = b_ref[...] - m_ref[...] * scale
        y = jnp.maximum(s * scale + shift, 0.0)
        o_ref[...] = jnp.dot(y, w_ref[...],
                             preferred_element_type=jnp.float32,
                             precision=lax.Precision.HIGHEST)

    return pl.pallas_call(
        body, out_shape=jax.ShapeDtypeStruct((N, D), jnp.float32))(
            parts, gamma, beta, mean, var, w)


def _tc_log_softmax(parts):
    def body(p_ref, o_ref):
        s = p_ref[0] + p_ref[1]
        m = jnp.max(s, axis=-1, keepdims=True)
        e = jnp.exp(s - m)
        lse = jnp.log(jnp.sum(e, axis=-1, keepdims=True)) + m
        o_ref[...] = s - lse

    return pl.pallas_call(
        body, out_shape=jax.ShapeDtypeStruct((N, D), jnp.float32))(parts)


FAST_FRAC = 0.64


def _pad_edges(edge_index, fast_core):
    e = edge_index.shape[1]
    cht = -(-e // (NS * K))
    chf = int(round(cht * FAST_FRAC))
    chs = cht - chf
    ch0, ch1 = (chf, chs) if fast_core == 0 else (chs, chf)
    ch = max(ch0, ch1)
    src = edge_index[0].astype(jnp.int32)
    dst = edge_index[1].astype(jnp.int32)
    pad = NS * cht * K - e
    src = jnp.concatenate([src, jnp.full((pad,), N, jnp.int32)])
    dst = jnp.concatenate([dst, jnp.zeros((pad,), jnp.int32)])

    def per_core(a, padval):
        p0 = a[:NS * ch0 * K].reshape(NS, ch0, K)
        p1 = a[NS * ch0 * K:].reshape(NS, ch1, K)
        p0 = jnp.pad(p0, ((0, 0), (0, ch - ch0), (0, 0)),
                     constant_values=padval)
        p1 = jnp.pad(p1, ((0, 0), (0, ch - ch1), (0, 0)),
                     constant_values=padval)
        return jnp.concatenate([p0, p1], axis=0)

    sd = jnp.stack([per_core(src, N), per_core(dst, 0)],
                   axis=1).reshape(NW * 2, ch, K)
    return sd, ch, ch0, ch1


def kernel(x, edge_index0, edge_index1, W0, W1, bn_gamma, bn_beta, bn_mean,
           bn_var):
    x = x.astype(jnp.float32)
    zrows = jnp.zeros((RCH, D), jnp.float32)
    zpad = jnp.zeros((HPAD, D), jnp.float32)
    g = bn_gamma.reshape(1, D)
    b = bn_beta.reshape(1, D)
    m = bn_mean.reshape(1, D)
    v = bn_var.reshape(1, D)

    FAST_CORE = 0
    sd0, cha, cha0, cha1 = _pad_edges(edge_index0, FAST_CORE)
    sd1, chb, chb0, chb1 = _pad_edges(edge_index1, FAST_CORE)

    h0 = jnp.concatenate([_tc_matmul(x, W0), zpad])
    p0 = _seg_sum_partials(h0, sd0, zrows, cha, cha0, cha1)
    h1 = jnp.concatenate([_tc_bn_relu_matmul(p0, g, b, m, v, W1), zpad])
    p1 = _seg_sum_partials(h1, sd1, zrows, chb, chb0, chb1)
    return _tc_log_softmax(p1)


# speedup vs baseline: 1.0198x; 1.0198x over previous
import functools

import jax
import jax.numpy as jnp
from jax import lax
from jax.experimental import pallas as pl
from jax.experimental.pallas import tpu as pltpu
from jax.experimental.pallas import tpu_sc as plsc

N = 10000
D = 128
EPS = 1e-5

NC = 2
NS = 16
NW = NC * NS
K = 128
RCH = 80
NRCH = N // RCH
HPAD = 16


GB = 1


def _seg_sum_partials(h, sd_pairs, zrows, ch, ch0, ch1):
    mesh = plsc.VectorSubcoreMesh(core_axis_name="c", subcore_axis_name="s",
                                  num_cores=NC, num_subcores=NS)

    @functools.partial(
        pl.kernel,
        out_type=jax.ShapeDtypeStruct((NC, N, D), jnp.float32),
        mesh=mesh,
        scratch_types=[
            pltpu.VMEM((K, D), jnp.float32),
            pltpu.VMEM((2, ch, K), jnp.int32),
            pltpu.VMEM_SHARED((N, D), jnp.float32),
            pltpu.SemaphoreType.DMA,
        ],
    )
    def k(h_hbm, sd_hbm, z_hbm, out_hbm, rows, sd, acc, gsem):
        cid = lax.axis_index("c")
        sid = lax.axis_index("s")
        wid = cid * NS + sid

        pltpu.sync_copy(sd_hbm.at[wid * 2], sd.at[0])
        pltpu.sync_copy(sd_hbm.at[wid * 2 + 1], sd.at[1])

        @pl.loop(sid, NRCH, step=NS)
        def _(q):
            pltpu.sync_copy(z_hbm, acc.at[pl.ds(q * RCH, RCH)])

        plsc.subcore_barrier()
        chw = jnp.where(cid == 0, ch0, ch1)
        pltpu.async_copy(h_hbm.at[sd.at[0, 0]], rows, gsem)

        @pl.loop(0, chw)
        def _(i):
            pltpu.make_async_copy(h_hbm.at[sd.at[0, i]], rows, gsem).wait()
            pltpu.sync_copy(rows, acc.at[sd.at[1, i]], add=True)

            @pl.when(i + 1 < chw)
            def _():
                pltpu.async_copy(h_hbm.at[sd.at[0, i + 1]], rows, gsem)

        plsc.subcore_barrier()

        @pl.loop(sid, NRCH, step=NS)
        def _(q):
            pltpu.sync_copy(acc.at[pl.ds(q * RCH, RCH)],
                            out_hbm.at[cid, pl.ds(q * RCH, RCH)])

    return k(h, sd_pairs, zrows)


def _tc_matmul(x, w):
    def body(x_ref, w_ref, o_ref):
        o_ref[...] = jnp.dot(x_ref[...], w_ref[...],
                             preferred_element_type=jnp.float32,
                             precision=lax.Precision.HIGHEST)

    return pl.pallas_call(
        body, out_shape=jax.ShapeDtypeStruct((N, D), jnp.float32))(x, w)


def _tc_bn_relu_matmul(parts, gamma, beta, mean, var, w):
    def body(p_ref, g_ref, b_ref, m_ref, v_ref, w_ref, o_ref):
        s = p_ref[0] + p_ref[1]
        scale = g_ref[...] * lax.rsqrt(v_ref[...] + EPS)
        shift = b_ref[...] - m_ref[...] * scale
        y = jnp.maximum(s * scale + shift, 0.0)
        o_ref[...] = jnp.dot(y, w_ref[...],
                             preferred_element_type=jnp.float32,
                             precision=lax.Precision.HIGHEST)

    return pl.pallas_call(
        body, out_shape=jax.ShapeDtypeStruct((N, D), jnp.float32))(
            parts, gamma, beta, mean, var, w)


def _tc_log_softmax(parts):
    def body(p_ref, o_ref):
        s = p_ref[0] + p_ref[1]
        m = jnp.max(s, axis=-1, keepdims=True)
        e = jnp.exp(s - m)
        lse = jnp.log(jnp.sum(e, axis=-1, keepdims=True)) + m
        o_ref[...] = s - lse

    return pl.pallas_call(
        body, out_shape=jax.ShapeDtypeStruct((N, D), jnp.float32))(parts)


FAST_FRAC = 0.58


def _pad_edges(edge_index, fast_core):
    e = edge_index.shape[1]
    cht = -(-e // (NS * K))
    chf = int(round(cht * FAST_FRAC))
    chs = cht - chf
    ch0, ch1 = (chf, chs) if fast_core == 0 else (chs, chf)
    ch = max(ch0, ch1)
    src = edge_index[0].astype(jnp.int32)
    dst = edge_index[1].astype(jnp.int32)
    pad = NS * cht * K - e
    src = jnp.concatenate([src, jnp.full((pad,), N, jnp.int32)])
    dst = jnp.concatenate([dst, jnp.zeros((pad,), jnp.int32)])

    def per_core(a, padval):
        p0 = a[:NS * ch0 * K].reshape(NS, ch0, K)
        p1 = a[NS * ch0 * K:].reshape(NS, ch1, K)
        p0 = jnp.pad(p0, ((0, 0), (0, ch - ch0), (0, 0)),
                     constant_values=padval)
        p1 = jnp.pad(p1, ((0, 0), (0, ch - ch1), (0, 0)),
                     constant_values=padval)
        return jnp.concatenate([p0, p1], axis=0)

    sd = jnp.stack([per_core(src, N), per_core(dst, 0)],
                   axis=1).reshape(NW * 2, ch, K)
    return sd, ch, ch0, ch1


def kernel(x, edge_index0, edge_index1, W0, W1, bn_gamma, bn_beta, bn_mean,
           bn_var):
    x = x.astype(jnp.float32)
    zrows = jnp.zeros((RCH, D), jnp.float32)
    zpad = jnp.zeros((HPAD, D), jnp.float32)
    g = bn_gamma.reshape(1, D)
    b = bn_beta.reshape(1, D)
    m = bn_mean.reshape(1, D)
    v = bn_var.reshape(1, D)

    FAST_CORE = 0
    sd0, cha, cha0, cha1 = _pad_edges(edge_index0, FAST_CORE)
    sd1, chb, chb0, chb1 = _pad_edges(edge_index1, FAST_CORE)

    h0 = jnp.concatenate([_tc_matmul(x, W0), zpad])
    p0 = _seg_sum_partials(h0, sd0, zrows, cha, cha0, cha1)
    h1 = jnp.concatenate([_tc_bn_relu_matmul(p0, g, b, m, v, W1), zpad])
    p1 = _seg_sum_partials(h1, sd1, zrows, chb, chb0, chb1)
    return _tc_log_softmax(p1)


# speedup vs baseline: 1.0484x; 1.0281x over previous
import functools

import jax
import jax.numpy as jnp
from jax import lax
from jax.experimental import pallas as pl
from jax.experimental.pallas import tpu as pltpu
from jax.experimental.pallas import tpu_sc as plsc

N = 10000
D = 128
EPS = 1e-5

NC = 2
NS = 16
NW = NC * NS
K = 128
RCH = 80
NRCH = N // RCH
HPAD = 16


GB = 1


def _seg_sum_partials(h, sd_pairs, zrows, ch, ch0, ch1):
    mesh = plsc.VectorSubcoreMesh(core_axis_name="c", subcore_axis_name="s",
                                  num_cores=NC, num_subcores=NS)

    @functools.partial(
        pl.kernel,
        out_type=jax.ShapeDtypeStruct((NC, N, D), jnp.float32),
        mesh=mesh,
        scratch_types=[
            pltpu.VMEM((K, D), jnp.float32),
            pltpu.VMEM((2, ch, K), jnp.int32),
            pltpu.VMEM_SHARED((N, D), jnp.float32),
            pltpu.SemaphoreType.DMA,
        ],
    )
    def k(h_hbm, sd_hbm, z_hbm, out_hbm, rows, sd, acc, gsem):
        cid = lax.axis_index("c")
        sid = lax.axis_index("s")
        wid = cid * NS + sid

        pltpu.sync_copy(sd_hbm.at[wid * 2], sd.at[0])
        pltpu.sync_copy(sd_hbm.at[wid * 2 + 1], sd.at[1])

        @pl.loop(sid, NRCH, step=NS)
        def _(q):
            pltpu.sync_copy(z_hbm, acc.at[pl.ds(q * RCH, RCH)])

        plsc.subcore_barrier()
        chw = jnp.where(cid == 0, ch0, ch1)
        pltpu.async_copy(h_hbm.at[sd.at[0, 0]], rows, gsem)

        @pl.loop(0, chw)
        def _(i):
            pltpu.make_async_copy(h_hbm.at[sd.at[0, i]], rows, gsem).wait()
            pltpu.sync_copy(rows, acc.at[sd.at[1, i]], add=True)

            @pl.when(i + 1 < chw)
            def _():
                pltpu.async_copy(h_hbm.at[sd.at[0, i + 1]], rows, gsem)

        plsc.subcore_barrier()

        @pl.loop(sid, NRCH, step=NS)
        def _(q):
            pltpu.sync_copy(acc.at[pl.ds(q * RCH, RCH)],
                            out_hbm.at[cid, pl.ds(q * RCH, RCH)])

    return k(h, sd_pairs, zrows)


def _tc_matmul(x, w):
    def body(x_ref, w_ref, o_ref):
        o_ref[...] = jnp.dot(x_ref[...], w_ref[...],
                             preferred_element_type=jnp.float32,
                             precision=lax.Precision.HIGHEST)

    return pl.pallas_call(
        body, out_shape=jax.ShapeDtypeStruct((N, D), jnp.float32))(x, w)


def _tc_bn_relu_matmul(parts, gamma, beta, mean, var, w):
    def body(p_ref, g_ref, b_ref, m_ref, v_ref, w_ref, o_ref):
        s = p_ref[0] + p_ref[1]
        scale = g_ref[...] * lax.rsqrt(v_ref[...] + EPS)
        shift = b_ref[...] - m_ref[...] * scale
        y = jnp.maximum(s * scale + shift, 0.0)
        o_ref[...] = jnp.dot(y, w_ref[...],
                             preferred_element_type=jnp.float32,
                             precision=lax.Precision.HIGHEST)

    return pl.pallas_call(
        body, out_shape=jax.ShapeDtypeStruct((N, D), jnp.float32))(
            parts, gamma, beta, mean, var, w)


def _tc_log_softmax(parts):
    def body(p_ref, o_ref):
        s = p_ref[0] + p_ref[1]
        m = jnp.max(s, axis=-1, keepdims=True)
        e = jnp.exp(s - m)
        lse = jnp.log(jnp.sum(e, axis=-1, keepdims=True)) + m
        o_ref[...] = s - lse

    return pl.pallas_call(
        body, out_shape=jax.ShapeDtypeStruct((N, D), jnp.float32))(parts)


FAST_FRAC = 0.61


def _pad_edges(edge_index, fast_core):
    e = edge_index.shape[1]
    cht = -(-e // (NS * K))
    chf = int(round(cht * FAST_FRAC))
    chs = cht - chf
    ch0, ch1 = (chf, chs) if fast_core == 0 else (chs, chf)
    ch = max(ch0, ch1)
    src = edge_index[0].astype(jnp.int32)
    dst = edge_index[1].astype(jnp.int32)
    pad = NS * cht * K - e
    src = jnp.concatenate([src, jnp.full((pad,), N, jnp.int32)])
    dst = jnp.concatenate([dst, jnp.zeros((pad,), jnp.int32)])

    def per_core(a, padval):
        p0 = a[:NS * ch0 * K].reshape(NS, ch0, K)
        p1 = a[NS * ch0 * K:].reshape(NS, ch1, K)
        p0 = jnp.pad(p0, ((0, 0), (0, ch - ch0), (0, 0)),
                     constant_values=padval)
        p1 = jnp.pad(p1, ((0, 0), (0, ch - ch1), (0, 0)),
                     constant_values=padval)
        return jnp.concatenate([p0, p1], axis=0)

    sd = jnp.stack([per_core(src, N), per_core(dst, 0)],
                   axis=1).reshape(NW * 2, ch, K)
    return sd, ch, ch0, ch1


def kernel(x, edge_index0, edge_index1, W0, W1, bn_gamma, bn_beta, bn_mean,
           bn_var):
    x = x.astype(jnp.float32)
    zrows = jnp.zeros((RCH, D), jnp.float32)
    zpad = jnp.zeros((HPAD, D), jnp.float32)
    g = bn_gamma.reshape(1, D)
    b = bn_beta.reshape(1, D)
    m = bn_mean.reshape(1, D)
    v = bn_var.reshape(1, D)

    FAST_CORE = 0
    sd0, cha, cha0, cha1 = _pad_edges(edge_index0, FAST_CORE)
    sd1, chb, chb0, chb1 = _pad_edges(edge_index1, FAST_CORE)

    h0 = jnp.concatenate([_tc_matmul(x, W0), zpad])
    p0 = _seg_sum_partials(h0, sd0, zrows, cha, cha0, cha1)
    h1 = jnp.concatenate([_tc_bn_relu_matmul(p0, g, b, m, v, W1), zpad])
    p1 = _seg_sum_partials(h1, sd1, zrows, chb, chb0, chb1)
    return _tc_log_softmax(p1)


# speedup vs baseline: 1.0815x; 1.0315x over previous
import functools

import jax
import jax.numpy as jnp
from jax import lax
from jax.experimental import pallas as pl
from jax.experimental.pallas import tpu as pltpu
from jax.experimental.pallas import tpu_sc as plsc

N = 10000
D = 128
EPS = 1e-5

NC = 2
NS = 16
NW = NC * NS
K = 128
RCH = 400
NRCH = N // RCH
HPAD = 16


GB = 1


def _seg_sum_partials(h, sd_pairs, zrows, ch, ch0, ch1):
    mesh = plsc.VectorSubcoreMesh(core_axis_name="c", subcore_axis_name="s",
                                  num_cores=NC, num_subcores=NS)

    @functools.partial(
        pl.kernel,
        out_type=jax.ShapeDtypeStruct((NC, N, D), jnp.float32),
        mesh=mesh,
        scratch_types=[
            pltpu.VMEM((K, D), jnp.float32),
            pltpu.VMEM((2, ch, K), jnp.int32),
            pltpu.VMEM_SHARED((N, D), jnp.float32),
            pltpu.SemaphoreType.DMA,
        ],
    )
    def k(h_hbm, sd_hbm, z_hbm, out_hbm, rows, sd, acc, gsem):
        cid = lax.axis_index("c")
        sid = lax.axis_index("s")
        wid = cid * NS + sid

        pltpu.sync_copy(sd_hbm.at[wid * 2], sd.at[0])
        pltpu.sync_copy(sd_hbm.at[wid * 2 + 1], sd.at[1])

        @pl.loop(sid, NRCH, step=NS)
        def _(q):
            pltpu.sync_copy(z_hbm, acc.at[pl.ds(q * RCH, RCH)])

        plsc.subcore_barrier()
        chw = jnp.where(cid == 0, ch0, ch1)
        pltpu.async_copy(h_hbm.at[sd.at[0, 0]], rows, gsem)

        @pl.loop(0, chw)
        def _(i):
            pltpu.make_async_copy(h_hbm.at[sd.at[0, i]], rows, gsem).wait()
            pltpu.sync_copy(rows, acc.at[sd.at[1, i]], add=True)

            @pl.when(i + 1 < chw)
            def _():
                pltpu.async_copy(h_hbm.at[sd.at[0, i + 1]], rows, gsem)

        plsc.subcore_barrier()

        @pl.loop(sid, NRCH, step=NS)
        def _(q):
            pltpu.sync_copy(acc.at[pl.ds(q * RCH, RCH)],
                            out_hbm.at[cid, pl.ds(q * RCH, RCH)])

    return k(h, sd_pairs, zrows)


def _tc_matmul(x, w):
    def body(x_ref, w_ref, o_ref):
        o_ref[...] = jnp.dot(x_ref[...], w_ref[...],
                             preferred_element_type=jnp.float32,
                             precision=lax.Precision.HIGHEST)

    return pl.pallas_call(
        body, out_shape=jax.ShapeDtypeStruct((N, D), jnp.float32))(x, w)


def _tc_bn_relu_matmul(parts, gamma, beta, mean, var, w):
    def body(p_ref, g_ref, b_ref, m_ref, v_ref, w_ref, o_ref):
        s = p_ref[0] + p_ref[1]
        scale = g_ref[...] * lax.rsqrt(v_ref[...] + EPS)
        shift = b_ref[...] - m_ref[...] * scale
        y = jnp.maximum(s * scale + shift, 0.0)
        o_ref[...] = jnp.dot(y, w_ref[...],
                             preferred_element_type=jnp.float32,
                             precision=lax.Precision.HIGHEST)

    return pl.pallas_call(
        body, out_shape=jax.ShapeDtypeStruct((N, D), jnp.float32))(
            parts, gamma, beta, mean, var, w)


def _tc_log_softmax(parts):
    def body(p_ref, o_ref):
        s = p_ref[0] + p_ref[1]
        m = jnp.max(s, axis=-1, keepdims=True)
        e = jnp.exp(s - m)
        lse = jnp.log(jnp.sum(e, axis=-1, keepdims=True)) + m
        o_ref[...] = s - lse

    return pl.pallas_call(
        body, out_shape=jax.ShapeDtypeStruct((N, D), jnp.float32))(parts)


FAST_FRAC = 0.61


def _pad_edges(edge_index, fast_core):
    e = edge_index.shape[1]
    cht = -(-e // (NS * K))
    chf = int(round(cht * FAST_FRAC))
    chs = cht - chf
    ch0, ch1 = (chf, chs) if fast_core == 0 else (chs, chf)
    ch = max(ch0, ch1)
    src = edge_index[0].astype(jnp.int32)
    dst = edge_index[1].astype(jnp.int32)
    pad = NS * cht * K - e
    src = jnp.concatenate([src, jnp.full((pad,), N, jnp.int32)])
    dst = jnp.concatenate([dst, jnp.zeros((pad,), jnp.int32)])

    def per_core(a, padval):
        p0 = a[:NS * ch0 * K].reshape(NS, ch0, K)
        p1 = a[NS * ch0 * K:].reshape(NS, ch1, K)
        p0 = jnp.pad(p0, ((0, 0), (0, ch - ch0), (0, 0)),
                     constant_values=padval)
        p1 = jnp.pad(p1, ((0, 0), (0, ch - ch1), (0, 0)),
                     constant_values=padval)
        return jnp.concatenate([p0, p1], axis=0)

    sd = jnp.stack([per_core(src, N), per_core(dst, 0)],
                   axis=1).reshape(NW * 2, ch, K)
    return sd, ch, ch0, ch1


def kernel(x, edge_index0, edge_index1, W0, W1, bn_gamma, bn_beta, bn_mean,
           bn_var):
    x = x.astype(jnp.float32)
    zrows = jnp.zeros((RCH, D), jnp.float32)
    zpad = jnp.zeros((HPAD, D), jnp.float32)
    g = bn_gamma.reshape(1, D)
    b = bn_beta.reshape(1, D)
    m = bn_mean.reshape(1, D)
    v = bn_var.reshape(1, D)

    FAST_CORE = 0
    sd0, cha, cha0, cha1 = _pad_edges(edge_index0, FAST_CORE)
    sd1, chb, chb0, chb1 = _pad_edges(edge_index1, FAST_CORE)

    h0 = jnp.concatenate([_tc_matmul(x, W0), zpad])
    p0 = _seg_sum_partials(h0, sd0, zrows, cha, cha0, cha1)
    h1 = jnp.concatenate([_tc_bn_relu_matmul(p0, g, b, m, v, W1), zpad])
    p1 = _seg_sum_partials(h1, sd1, zrows, chb, chb0, chb1)
    return _tc_log_softmax(p1)
